# Initial kernel scaffold; baseline (speedup 1.0000x reference)
import jax
import jax.numpy as jnp
from jax.experimental import pallas as pl


def kernel(tokenIndex, weights):
    raise NotImplementedError("write your pallas kernel here")


# speedup vs baseline: 1.6848x; 1.6848x over previous
import functools

import jax
import jax.numpy as jnp
from jax import lax
from jax.experimental import pallas as pl
from jax.experimental.pallas import tpu as pltpu
from jax.experimental.pallas import tpu_sc as plsc

_VOCAB = 1000000
_EMBED_DIM = 64
_BATCH = 16384
_HIST = 50
_TOTAL = _BATCH * _HIST

_NC = 2
_NS = 16
_NW = _NC * _NS
_PER_W = _TOTAL // _NW
_CHUNK = 128
_ITERS = _PER_W // _CHUNK


def _embed_kernel(idx_hbm, table_hbm, out_hbm, idx_v, rows_v, sem):
    wid = lax.axis_index("s") * _NC + lax.axis_index("c")
    base = wid * _PER_W
    pltpu.sync_copy(idx_hbm.at[wid], idx_v)

    def step(i, carry):
        pltpu.async_copy(table_hbm.at[idx_v.at[i]], rows_v, sem).wait()
        pltpu.sync_copy(rows_v, out_hbm.at[pl.ds(base + i * _CHUNK, _CHUNK)])
        return carry

    lax.fori_loop(0, _ITERS, step, 0)


@jax.jit
def _embed(idx3, weights):
    mesh = plsc.VectorSubcoreMesh(core_axis_name="c", subcore_axis_name="s")
    f = functools.partial(
        pl.kernel,
        mesh=mesh,
        out_type=jax.ShapeDtypeStruct((_TOTAL, _EMBED_DIM), jnp.float32),
        scratch_types=[
            pltpu.VMEM((_ITERS, _CHUNK), jnp.int32),
            pltpu.VMEM((_CHUNK, _EMBED_DIM), jnp.float32),
            pltpu.SemaphoreType.DMA,
        ],
        compiler_params=pltpu.CompilerParams(use_tc_tiling_on_sc=False),
    )(_embed_kernel)
    return f(idx3, weights)


def kernel(tokenIndex, weights):
    idx3 = tokenIndex.astype(jnp.int32).reshape(_NW, _ITERS, _CHUNK)
    out = _embed(idx3, weights)
    return out.reshape(_BATCH, _HIST, _EMBED_DIM)


# speedup vs baseline: 1.8712x; 1.1106x over previous
import functools

import jax
import jax.numpy as jnp
from jax import lax
from jax.experimental import pallas as pl
from jax.experimental.pallas import tpu as pltpu
from jax.experimental.pallas import tpu_sc as plsc

_VOCAB = 1000000
_EMBED_DIM = 64
_BATCH = 16384
_HIST = 50
_TOTAL = _BATCH * _HIST

_NC = 2
_NS = 16
_NW = _NC * _NS
_PER_W = _TOTAL // _NW
_CHUNK = 128
_K = 5
_SUPER = _K * _CHUNK
_NSUP = _PER_W // _SUPER
_NCHUNK = _PER_W // _CHUNK


def _fire(table_hbm, idx_v, buf, sem, sup):
    for j in range(_K):
        pltpu.async_copy(
            table_hbm.at[idx_v.at[sup * _K + j]],
            buf.at[pl.ds(j * _CHUNK, _CHUNK)],
            sem,
        )


def _embed_kernel(idx_hbm, table_hbm, out_hbm, idx_v, buf0, buf1, sem0, sem1):
    wid = lax.axis_index("s") * _NC + lax.axis_index("c")
    base = wid * _PER_W
    pltpu.sync_copy(idx_hbm.at[wid], idx_v)

    bufs = (buf0, buf1)
    sems = (sem0, sem1)

    _fire(table_hbm, idx_v, buf0, sem0, 0)
    _fire(table_hbm, idx_v, buf1, sem1, 1)

    def step(s2, carry):
        for b in range(2):
            sup = s2 * 2 + b
            buf, sem = bufs[b], sems[b]
            pltpu.make_async_copy(
                out_hbm.at[pl.ds(0, _SUPER)], buf, sem
            ).wait()
            pltpu.sync_copy(buf, out_hbm.at[pl.ds(base + sup * _SUPER, _SUPER)])

            @pl.when(sup + 2 < _NSUP)
            def _():
                _fire(table_hbm, idx_v, buf, sem, sup + 2)

        return carry

    lax.fori_loop(0, _NSUP // 2, step, 0)


@jax.jit
def _embed(idx3, weights):
    mesh = plsc.VectorSubcoreMesh(core_axis_name="c", subcore_axis_name="s")
    f = functools.partial(
        pl.kernel,
        mesh=mesh,
        out_type=jax.ShapeDtypeStruct((_TOTAL, _EMBED_DIM), jnp.float32),
        scratch_types=[
            pltpu.VMEM((_NCHUNK, _CHUNK), jnp.int32),
            pltpu.VMEM((_SUPER, _EMBED_DIM), jnp.float32),
            pltpu.VMEM((_SUPER, _EMBED_DIM), jnp.float32),
            pltpu.SemaphoreType.DMA,
            pltpu.SemaphoreType.DMA,
        ],
        compiler_params=pltpu.CompilerParams(use_tc_tiling_on_sc=False),
    )(_embed_kernel)
    return f(idx3, weights)


def kernel(tokenIndex, weights):
    idx3 = tokenIndex.astype(jnp.int32).reshape(_NW, _NCHUNK, _CHUNK)
    out = _embed(idx3, weights)
    return out.reshape(_BATCH, _HIST, _EMBED_DIM)
